# Initial kernel scaffold; baseline (speedup 1.0000x reference)
import jax
import jax.numpy as jnp
from jax.experimental import pallas as pl


def kernel(op_idx, a, b, c, op_embed, W_in, b_in, Wr, br, Wk, bk, V, W1, b1, W2, b2):
    raise NotImplementedError("write your pallas kernel here")


# speedup vs baseline: 1.4167x; 1.4167x over previous
import functools

import jax
import jax.numpy as jnp
from jax import lax
from jax.experimental import pallas as pl
from jax.experimental.pallas import tpu as pltpu
from jax.experimental.pallas import tpu_sc as plsc

D_MODEL = 1024
NUM_TILES = 16
K_TOP = 4
TABLE_SIZE = 2048
N_OPS = 8
B = 8192
TERNARY_W = 0.01
SPARSITY_W = 0.005
DIVERSITY_W = 0.01

BLK = 512
GRID = B // BLK

NC = 2
NS = 16
L = 16
NW = NC * NS
TOK_W = B // NW
CH = 16
NCH = TOK_W // CH
ROWS = CH * K_TOP


def _front_kernel(op_ref, a_ref, b_ref, c_ref,
                  op_embed_ref, w_op_ref, w_a_ref, w_b_ref, w_c_ref,
                  b_in_ref, wr_ref, br_ref, wk_ref, bk_ref, v_ref,
                  h_ref, tile_ref, flat_ref, gates_ref,
                  psum_ref, csum_ref, esum_ref, tsum_ref):
    i = pl.program_id(0)

    op_col = op_ref[...]
    a_col = a_ref[...]
    b_col = b_ref[...]
    c_col = c_ref[...].astype(jnp.float32)

    iota8 = lax.broadcasted_iota(jnp.int32, (BLK, 8), 1)
    oh = (op_col == iota8).astype(jnp.float32)
    a_bits = ((a_col >> iota8) & 1).astype(jnp.float32)
    b_bits = ((b_col >> iota8) & 1).astype(jnp.float32)

    bf = jnp.bfloat16
    op_emb = jnp.dot(oh.astype(bf), op_embed_ref[...].astype(bf),
                     preferred_element_type=jnp.float32)
    h = jnp.dot(op_emb.astype(bf), w_op_ref[...].astype(bf),
                preferred_element_type=jnp.float32)
    h = h + jnp.dot(a_bits.astype(bf), w_a_ref[...].astype(bf),
                    preferred_element_type=jnp.float32)
    h = h + jnp.dot(b_bits.astype(bf), w_b_ref[...].astype(bf),
                    preferred_element_type=jnp.float32)
    h = h + c_col * w_c_ref[...].astype(bf).astype(jnp.float32)
    h = h + b_in_ref[...]
    h_ref[...] = h

    rl = jnp.dot(h.astype(bf), wr_ref[...].astype(bf),
                 preferred_element_type=jnp.float32) + br_ref[...]
    rm = jnp.max(rl, axis=1, keepdims=True)
    re = jnp.exp(rl - rm)
    p = re / jnp.sum(re, axis=1, keepdims=True)
    ent = -jnp.sum(p * jnp.log(p + 1e-9), axis=1, keepdims=True)

    NEG = -1e30
    iota16 = lax.broadcasted_iota(jnp.int32, (BLK, NUM_TILES), 1)
    cur = rl
    tis, tvs = [], []
    for _ in range(K_TOP):
        mv = jnp.max(cur, axis=1, keepdims=True)
        idx = jnp.min(jnp.where(cur == mv, iota16, NUM_TILES),
                      axis=1, keepdims=True)
        tis.append(idx)
        tvs.append(mv)
        cur = jnp.where(iota16 == idx, NEG, cur)
    tile_idx = jnp.concatenate(tis, axis=1)
    top_vals = jnp.concatenate(tvs, axis=1)
    ge = jnp.exp(top_vals - top_vals[:, :1])
    gates = ge / jnp.sum(ge, axis=1, keepdims=True)
    tile_ref[...] = tile_idx
    gates_ref[...] = gates

    kl = jnp.dot(h.astype(bf), wk_ref[...].astype(bf),
                 preferred_element_type=jnp.float32) + bk_ref[...]
    km = jnp.max(kl, axis=1, keepdims=True)
    iota2k = lax.broadcasted_iota(jnp.int32, (BLK, TABLE_SIZE), 1)
    key_idx = jnp.min(jnp.where(kl == km, iota2k, TABLE_SIZE),
                      axis=1, keepdims=True)
    flat_ref[...] = tile_idx * TABLE_SIZE + key_idx

    v = v_ref[0]
    av = jnp.abs(v)
    s = av * (av - 1.0)
    tern_part = jnp.sum(jnp.sum(s * s, axis=1, keepdims=True),
                        axis=0, keepdims=True)

    cnt = jnp.sum((cur == NEG).astype(jnp.float32), axis=0, keepdims=True)

    @pl.when(i == 0)
    def _():
        psum_ref[...] = jnp.zeros_like(psum_ref)
        csum_ref[...] = jnp.zeros_like(csum_ref)
        esum_ref[...] = jnp.zeros_like(esum_ref)
        tsum_ref[...] = jnp.zeros_like(tsum_ref)

    psum_ref[...] += jnp.sum(p, axis=0, keepdims=True)
    csum_ref[...] += cnt
    esum_ref[...] += jnp.sum(ent, axis=0, keepdims=True)
    tsum_ref[...] += tern_part


def _head_kernel(h_ref, ws_ref, w1_ref, b1_ref, w2_ref, b2_ref, out_ref):
    ffn = h_ref[...] + ws_ref[...]
    bf = jnp.bfloat16
    hid = jnp.maximum(
        jnp.dot(ffn.astype(bf), w1_ref[...].astype(bf),
                preferred_element_type=jnp.float32)
        + b1_ref[...], 0.0)
    res = jnp.dot(hid.astype(bf), w2_ref[...].astype(bf),
                  preferred_element_type=jnp.float32) + b2_ref[...]
    out_ref[...] = 1.0 / (1.0 + jnp.exp(-res))


def _sc_gather_ws(fi_hbm, gb_hbm, tab_hbm, ws_hbm,
                  idx_v, g_v, rows_v, acc_v, sem):
    wid = lax.axis_index("s") * NC + lax.axis_index("c")
    tok_base = wid * TOK_W

    def chunk_body(ci, _c):
        tok0 = tok_base + ci * CH
        pltpu.sync_copy(fi_hbm.at[pl.ds(tok0 * K_TOP, ROWS)], idx_v)
        pltpu.async_copy(tab_hbm.at[idx_v], rows_v, sem).wait()
        pltpu.sync_copy(gb_hbm.at[pl.ds(tok0 * K_TOP * L, ROWS * L)], g_v)

        def tok_body(t, _t):
            r = t * K_TOP
            g0 = g_v[pl.ds((r + 0) * L, L)]
            g1 = g_v[pl.ds((r + 1) * L, L)]
            g2 = g_v[pl.ds((r + 2) * L, L)]
            g3 = g_v[pl.ds((r + 3) * L, L)]

            def d_body(j, _d):
                sl = pl.ds(j * L, L)
                acc_v[t, sl] = (g0 * rows_v[r, sl]
                                + g1 * rows_v[r + 1, sl]
                                + g2 * rows_v[r + 2, sl]
                                + g3 * rows_v[r + 3, sl])
                return 0

            lax.fori_loop(0, D_MODEL // L, d_body, 0)
            return 0

        lax.fori_loop(0, CH, tok_body, 0)
        pltpu.sync_copy(acc_v, ws_hbm.at[pl.ds(tok0, CH)])
        return 0

    lax.fori_loop(0, NCH, chunk_body, 0)


def _sc_ws(fi, gb, tab):
    return pl.kernel(
        _sc_gather_ws,
        out_type=jax.ShapeDtypeStruct((B, D_MODEL), jnp.float32),
        mesh=plsc.VectorSubcoreMesh(core_axis_name="c", subcore_axis_name="s"),
        scratch_types=[
            pltpu.VMEM((ROWS,), jnp.int32),
            pltpu.VMEM((ROWS * L,), jnp.float32),
            pltpu.VMEM((ROWS, D_MODEL), jnp.float32),
            pltpu.VMEM((CH, D_MODEL), jnp.float32),
            pltpu.SemaphoreType.DMA,
        ],
    )(fi, gb, tab)


def kernel(op_idx, a, b, c, op_embed, W_in, b_in, Wr, br, Wk, bk, V,
           W1, b1, W2, b2):
    op2 = op_idx.astype(jnp.int32)[:, None]
    a2 = a.astype(jnp.int32)[:, None]
    b2c = b.astype(jnp.int32)[:, None]
    c2 = c.astype(jnp.int32)[:, None]

    w_op = W_in[0:32]
    w_a = W_in[32:40]
    w_b = W_in[40:48]
    w_c = W_in[48:49]

    const2 = lambda shape: pl.BlockSpec(shape, lambda i: (0, 0))
    h, tile_idx, flat, gates, psum, csum, esum, tsum = pl.pallas_call(
        _front_kernel,
        grid=(GRID,),
        in_specs=[
            pl.BlockSpec((BLK, 1), lambda i: (i, 0)),
            pl.BlockSpec((BLK, 1), lambda i: (i, 0)),
            pl.BlockSpec((BLK, 1), lambda i: (i, 0)),
            pl.BlockSpec((BLK, 1), lambda i: (i, 0)),
            const2((N_OPS, 32)),
            const2((32, D_MODEL)),
            const2((8, D_MODEL)),
            const2((8, D_MODEL)),
            const2((1, D_MODEL)),
            const2((1, D_MODEL)),
            const2((D_MODEL, NUM_TILES)),
            const2((1, NUM_TILES)),
            const2((D_MODEL, TABLE_SIZE)),
            const2((1, TABLE_SIZE)),
            pl.BlockSpec((1, TABLE_SIZE, D_MODEL), lambda i: (i, 0, 0)),
        ],
        out_specs=[
            pl.BlockSpec((BLK, D_MODEL), lambda i: (i, 0)),
            pl.BlockSpec((BLK, K_TOP), lambda i: (i, 0)),
            pl.BlockSpec((BLK, K_TOP), lambda i: (i, 0)),
            pl.BlockSpec((BLK, K_TOP), lambda i: (i, 0)),
            const2((1, NUM_TILES)),
            const2((1, NUM_TILES)),
            const2((1, 1)),
            const2((1, 1)),
        ],
        out_shape=[
            jax.ShapeDtypeStruct((B, D_MODEL), jnp.float32),
            jax.ShapeDtypeStruct((B, K_TOP), jnp.int32),
            jax.ShapeDtypeStruct((B, K_TOP), jnp.int32),
            jax.ShapeDtypeStruct((B, K_TOP), jnp.float32),
            jax.ShapeDtypeStruct((1, NUM_TILES), jnp.float32),
            jax.ShapeDtypeStruct((1, NUM_TILES), jnp.float32),
            jax.ShapeDtypeStruct((1, 1), jnp.float32),
            jax.ShapeDtypeStruct((1, 1), jnp.float32),
        ],
        compiler_params=pltpu.CompilerParams(
            dimension_semantics=("arbitrary",)),
    )(op2, a2, b2c, c2, op_embed, w_op, w_a, w_b, w_c,
      b_in.reshape(1, D_MODEL), Wr, br.reshape(1, NUM_TILES),
      Wk, bk.reshape(1, TABLE_SIZE), V)

    fi = flat.reshape(B * K_TOP)
    gb = jnp.broadcast_to(gates[:, :, None], (B, K_TOP, L)).reshape(-1)
    tab = V.reshape(NUM_TILES * TABLE_SIZE, D_MODEL)
    ws = _sc_ws(fi, gb, tab)

    result = pl.pallas_call(
        _head_kernel,
        grid=(GRID,),
        in_specs=[
            pl.BlockSpec((BLK, D_MODEL), lambda i: (i, 0)),
            pl.BlockSpec((BLK, D_MODEL), lambda i: (i, 0)),
            const2((D_MODEL, 64)),
            const2((1, 64)),
            const2((64, 8)),
            const2((1, 8)),
        ],
        out_specs=pl.BlockSpec((BLK, 8), lambda i: (i, 0)),
        out_shape=jax.ShapeDtypeStruct((B, 8), jnp.float32),
    )(h, ws, W1, b1.reshape(1, 64), W2, b2.reshape(1, 8))

    total = float(NUM_TILES * TABLE_SIZE * D_MODEL)
    tern = TERNARY_W * (tsum[0, 0] / total)
    sparsity = SPARSITY_W * (esum[0, 0] / B)
    frac = csum[0] / B
    imp = psum[0] / B
    diversity = DIVERSITY_W * NUM_TILES * jnp.sum(frac * imp)
    aux = tern + sparsity + diversity
    return result, tile_idx, aux


# speedup vs baseline: 1.8117x; 1.2788x over previous
import functools

import jax
import jax.numpy as jnp
from jax import lax
from jax.experimental import pallas as pl
from jax.experimental.pallas import tpu as pltpu
from jax.experimental.pallas import tpu_sc as plsc

D_MODEL = 1024
NUM_TILES = 16
K_TOP = 4
TABLE_SIZE = 2048
N_OPS = 8
B = 8192
TERNARY_W = 0.01
SPARSITY_W = 0.005
DIVERSITY_W = 0.01

BLK = 512
GRID = B // BLK

NC = 2
NS = 16
L = 16
NW = NC * NS
TOK_W = B // NW
CH = 8
NCH = TOK_W // CH
ROWS = CH * K_TOP


def _front_kernel(op_ref, a_ref, b_ref, c_ref,
                  op_embed_ref, w_op_ref, w_a_ref, w_b_ref, w_c_ref,
                  b_in_ref, wr_ref, br_ref, wk_ref, bk_ref, v_ref,
                  h_ref, tile_ref, flat_ref, gates_ref,
                  psum_ref, csum_ref, esum_ref, tsum_ref):
    i = pl.program_id(0)

    op_col = op_ref[...]
    a_col = a_ref[...]
    b_col = b_ref[...]
    c_col = c_ref[...].astype(jnp.float32)

    iota8 = lax.broadcasted_iota(jnp.int32, (BLK, 8), 1)
    oh = (op_col == iota8).astype(jnp.float32)
    a_bits = ((a_col >> iota8) & 1).astype(jnp.float32)
    b_bits = ((b_col >> iota8) & 1).astype(jnp.float32)

    bf = jnp.bfloat16
    op_emb = jnp.dot(oh.astype(bf), op_embed_ref[...].astype(bf),
                     preferred_element_type=jnp.float32)
    h = jnp.dot(op_emb.astype(bf), w_op_ref[...].astype(bf),
                preferred_element_type=jnp.float32)
    h = h + jnp.dot(a_bits.astype(bf), w_a_ref[...].astype(bf),
                    preferred_element_type=jnp.float32)
    h = h + jnp.dot(b_bits.astype(bf), w_b_ref[...].astype(bf),
                    preferred_element_type=jnp.float32)
    h = h + c_col * w_c_ref[...].astype(bf).astype(jnp.float32)
    h = h + b_in_ref[...]
    h_ref[...] = h

    rl = jnp.dot(h.astype(bf), wr_ref[...].astype(bf),
                 preferred_element_type=jnp.float32) + br_ref[...]
    rm = jnp.max(rl, axis=1, keepdims=True)
    re = jnp.exp(rl - rm)
    p = re / jnp.sum(re, axis=1, keepdims=True)
    ent = -jnp.sum(p * jnp.log(p + 1e-9), axis=1, keepdims=True)

    NEG = -1e30
    iota16 = lax.broadcasted_iota(jnp.int32, (BLK, NUM_TILES), 1)
    cur = rl
    tis, tvs = [], []
    for _ in range(K_TOP):
        mv = jnp.max(cur, axis=1, keepdims=True)
        idx = jnp.min(jnp.where(cur == mv, iota16, NUM_TILES),
                      axis=1, keepdims=True)
        tis.append(idx)
        tvs.append(mv)
        cur = jnp.where(iota16 == idx, NEG, cur)
    tile_idx = jnp.concatenate(tis, axis=1)
    top_vals = jnp.concatenate(tvs, axis=1)
    ge = jnp.exp(top_vals - top_vals[:, :1])
    gates = ge / jnp.sum(ge, axis=1, keepdims=True)
    tile_ref[...] = tile_idx
    gates_ref[...] = gates

    kl = jnp.dot(h.astype(bf), wk_ref[...].astype(bf),
                 preferred_element_type=jnp.float32) + bk_ref[...]
    km = jnp.max(kl, axis=1, keepdims=True)
    iota2k = lax.broadcasted_iota(jnp.int32, (BLK, TABLE_SIZE), 1)
    key_idx = jnp.min(jnp.where(kl == km, iota2k, TABLE_SIZE),
                      axis=1, keepdims=True)
    flat_ref[...] = tile_idx * TABLE_SIZE + key_idx

    v = v_ref[0]
    av = jnp.abs(v)
    s = av * (av - 1.0)
    tern_part = jnp.sum(jnp.sum(s * s, axis=1, keepdims=True),
                        axis=0, keepdims=True)

    cnt = jnp.sum((cur == NEG).astype(jnp.float32), axis=0, keepdims=True)

    @pl.when(i == 0)
    def _():
        psum_ref[...] = jnp.zeros_like(psum_ref)
        csum_ref[...] = jnp.zeros_like(csum_ref)
        esum_ref[...] = jnp.zeros_like(esum_ref)
        tsum_ref[...] = jnp.zeros_like(tsum_ref)

    psum_ref[...] += jnp.sum(p, axis=0, keepdims=True)
    csum_ref[...] += cnt
    esum_ref[...] += jnp.sum(ent, axis=0, keepdims=True)
    tsum_ref[...] += tern_part


def _head_kernel(h_ref, ws_ref, w1_ref, b1_ref, w2_ref, b2_ref, out_ref):
    ffn = h_ref[...] + ws_ref[...]
    bf = jnp.bfloat16
    hid = jnp.maximum(
        jnp.dot(ffn.astype(bf), w1_ref[...].astype(bf),
                preferred_element_type=jnp.float32)
        + b1_ref[...], 0.0)
    res = jnp.dot(hid.astype(bf), w2_ref[...].astype(bf),
                  preferred_element_type=jnp.float32) + b2_ref[...]
    out_ref[...] = 1.0 / (1.0 + jnp.exp(-res))


def _sc_gather_ws(fi_hbm, gb_hbm, tab_hbm, ws_hbm,
                  idx0, idx1, g0v, g1v, rows0, rows1, acc0, acc1,
                  sem_a, sem_b, sem_o):
    wid = lax.axis_index("s") * NC + lax.axis_index("c")
    tok_base = wid * TOK_W

    def fire(ci, idx_ref, g_ref, rows_ref, sem):
        tok0 = tok_base + ci * CH
        pltpu.sync_copy(fi_hbm.at[pl.ds(tok0 * K_TOP, ROWS)], idx_ref)
        pltpu.async_copy(tab_hbm.at[idx_ref], rows_ref, sem)
        pltpu.sync_copy(gb_hbm.at[pl.ds(tok0 * K_TOP * L, ROWS * L)], g_ref)

    def drain_gather(rows_ref, sem):
        pltpu.make_async_copy(tab_hbm.at[pl.ds(0, ROWS)], rows_ref, sem).wait()

    def drain_store(acc_ref):
        pltpu.make_async_copy(ws_hbm.at[pl.ds(0, CH)], acc_ref, sem_o).wait()

    def compute(g_ref, rows_ref, acc_ref):
        def tok_body(t, _t):
            r = t * K_TOP
            ga = g_ref[pl.ds((r + 0) * L, L)]
            gb = g_ref[pl.ds((r + 1) * L, L)]
            gc = g_ref[pl.ds((r + 2) * L, L)]
            gd = g_ref[pl.ds((r + 3) * L, L)]

            @plsc.parallel_loop(0, D_MODEL // L, 1, unroll=4)
            def d_body(j):
                sl = pl.ds(j * L, L)
                acc_ref[t, sl] = (ga * rows_ref[r, sl]
                                  + gb * rows_ref[r + 1, sl]
                                  + gc * rows_ref[r + 2, sl]
                                  + gd * rows_ref[r + 3, sl])

            return 0

        lax.fori_loop(0, CH, tok_body, 0)

    fire(0, idx0, g0v, rows0, sem_a)

    def pair_body(cp, _p):
        c0 = cp * 2
        fire(c0 + 1, idx1, g1v, rows1, sem_b)
        drain_gather(rows0, sem_a)

        @pl.when(cp > 0)
        def _():
            drain_store(acc0)

        compute(g0v, rows0, acc0)
        pltpu.async_copy(acc0, ws_hbm.at[pl.ds(tok_base + c0 * CH, CH)], sem_o)

        @pl.when(cp + 1 < NCH // 2)
        def _():
            fire(c0 + 2, idx0, g0v, rows0, sem_a)

        drain_gather(rows1, sem_b)

        @pl.when(cp > 0)
        def _():
            drain_store(acc1)

        compute(g1v, rows1, acc1)
        pltpu.async_copy(acc1, ws_hbm.at[pl.ds(tok_base + (c0 + 1) * CH, CH)],
                         sem_o)
        return 0

    lax.fori_loop(0, NCH // 2, pair_body, 0)
    drain_store(acc0)
    drain_store(acc1)


def _sc_ws(fi, gb, tab):
    return pl.kernel(
        _sc_gather_ws,
        out_type=jax.ShapeDtypeStruct((B, D_MODEL), jnp.float32),
        mesh=plsc.VectorSubcoreMesh(core_axis_name="c", subcore_axis_name="s"),
        scratch_types=[
            pltpu.VMEM((ROWS,), jnp.int32),
            pltpu.VMEM((ROWS,), jnp.int32),
            pltpu.VMEM((ROWS * L,), jnp.float32),
            pltpu.VMEM((ROWS * L,), jnp.float32),
            pltpu.VMEM((ROWS, D_MODEL), jnp.float32),
            pltpu.VMEM((ROWS, D_MODEL), jnp.float32),
            pltpu.VMEM((CH, D_MODEL), jnp.float32),
            pltpu.VMEM((CH, D_MODEL), jnp.float32),
            pltpu.SemaphoreType.DMA,
            pltpu.SemaphoreType.DMA,
            pltpu.SemaphoreType.DMA,
        ],
    )(fi, gb, tab)


def kernel(op_idx, a, b, c, op_embed, W_in, b_in, Wr, br, Wk, bk, V,
           W1, b1, W2, b2):
    op2 = op_idx.astype(jnp.int32)[:, None]
    a2 = a.astype(jnp.int32)[:, None]
    b2c = b.astype(jnp.int32)[:, None]
    c2 = c.astype(jnp.int32)[:, None]

    w_op = W_in[0:32]
    w_a = W_in[32:40]
    w_b = W_in[40:48]
    w_c = W_in[48:49]

    const2 = lambda shape: pl.BlockSpec(shape, lambda i: (0, 0))
    h, tile_idx, flat, gates, psum, csum, esum, tsum = pl.pallas_call(
        _front_kernel,
        grid=(GRID,),
        in_specs=[
            pl.BlockSpec((BLK, 1), lambda i: (i, 0)),
            pl.BlockSpec((BLK, 1), lambda i: (i, 0)),
            pl.BlockSpec((BLK, 1), lambda i: (i, 0)),
            pl.BlockSpec((BLK, 1), lambda i: (i, 0)),
            const2((N_OPS, 32)),
            const2((32, D_MODEL)),
            const2((8, D_MODEL)),
            const2((8, D_MODEL)),
            const2((1, D_MODEL)),
            const2((1, D_MODEL)),
            const2((D_MODEL, NUM_TILES)),
            const2((1, NUM_TILES)),
            const2((D_MODEL, TABLE_SIZE)),
            const2((1, TABLE_SIZE)),
            pl.BlockSpec((1, TABLE_SIZE, D_MODEL), lambda i: (i, 0, 0)),
        ],
        out_specs=[
            pl.BlockSpec((BLK, D_MODEL), lambda i: (i, 0)),
            pl.BlockSpec((BLK, K_TOP), lambda i: (i, 0)),
            pl.BlockSpec((BLK, K_TOP), lambda i: (i, 0)),
            pl.BlockSpec((BLK, K_TOP), lambda i: (i, 0)),
            const2((1, NUM_TILES)),
            const2((1, NUM_TILES)),
            const2((1, 1)),
            const2((1, 1)),
        ],
        out_shape=[
            jax.ShapeDtypeStruct((B, D_MODEL), jnp.float32),
            jax.ShapeDtypeStruct((B, K_TOP), jnp.int32),
            jax.ShapeDtypeStruct((B, K_TOP), jnp.int32),
            jax.ShapeDtypeStruct((B, K_TOP), jnp.float32),
            jax.ShapeDtypeStruct((1, NUM_TILES), jnp.float32),
            jax.ShapeDtypeStruct((1, NUM_TILES), jnp.float32),
            jax.ShapeDtypeStruct((1, 1), jnp.float32),
            jax.ShapeDtypeStruct((1, 1), jnp.float32),
        ],
        compiler_params=pltpu.CompilerParams(
            dimension_semantics=("arbitrary",)),
    )(op2, a2, b2c, c2, op_embed, w_op, w_a, w_b, w_c,
      b_in.reshape(1, D_MODEL), Wr, br.reshape(1, NUM_TILES),
      Wk, bk.reshape(1, TABLE_SIZE), V)

    fi = flat.reshape(B * K_TOP)
    gb = jnp.broadcast_to(gates[:, :, None], (B, K_TOP, L)).reshape(-1)
    tab = V.reshape(NUM_TILES * TABLE_SIZE, D_MODEL)
    ws = _sc_ws(fi, gb, tab)

    result = pl.pallas_call(
        _head_kernel,
        grid=(GRID,),
        in_specs=[
            pl.BlockSpec((BLK, D_MODEL), lambda i: (i, 0)),
            pl.BlockSpec((BLK, D_MODEL), lambda i: (i, 0)),
            const2((D_MODEL, 64)),
            const2((1, 64)),
            const2((64, 8)),
            const2((1, 8)),
        ],
        out_specs=pl.BlockSpec((BLK, 8), lambda i: (i, 0)),
        out_shape=jax.ShapeDtypeStruct((B, 8), jnp.float32),
    )(h, ws, W1, b1.reshape(1, 64), W2, b2.reshape(1, 8))

    total = float(NUM_TILES * TABLE_SIZE * D_MODEL)
    tern = TERNARY_W * (tsum[0, 0] / total)
    sparsity = SPARSITY_W * (esum[0, 0] / B)
    frac = csum[0] / B
    imp = psum[0] / B
    diversity = DIVERSITY_W * NUM_TILES * jnp.sum(frac * imp)
    aux = tern + sparsity + diversity
    return result, tile_idx, aux


# speedup vs baseline: 1.8241x; 1.0068x over previous
import functools

import jax
import jax.numpy as jnp
from jax import lax
from jax.experimental import pallas as pl
from jax.experimental.pallas import tpu as pltpu
from jax.experimental.pallas import tpu_sc as plsc

D_MODEL = 1024
NUM_TILES = 16
K_TOP = 4
TABLE_SIZE = 2048
N_OPS = 8
B = 8192
TERNARY_W = 0.01
SPARSITY_W = 0.005
DIVERSITY_W = 0.01

BLK = 512
GRID = B // BLK

NC = 2
NS = 16
L = 16
NW = NC * NS
TOK_W = B // NW
CH = 8
NCH = TOK_W // CH
ROWS = CH * K_TOP


def _front_kernel(op_ref, a_ref, b_ref, c_ref,
                  op_embed_ref, w_op_ref, w_a_ref, w_b_ref, w_c_ref,
                  b_in_ref, wr_ref, br_ref, wk_ref, bk_ref, v_ref,
                  h_ref, tile_ref, flat_ref, gates_ref,
                  psum_ref, csum_ref, esum_ref, tsum_ref):
    i = pl.program_id(0)

    op_col = op_ref[...]
    a_col = a_ref[...]
    b_col = b_ref[...]
    c_col = c_ref[...].astype(jnp.float32)

    iota8 = lax.broadcasted_iota(jnp.int32, (BLK, 8), 1)
    oh = (op_col == iota8).astype(jnp.float32)
    a_bits = ((a_col >> iota8) & 1).astype(jnp.float32)
    b_bits = ((b_col >> iota8) & 1).astype(jnp.float32)

    bf = jnp.bfloat16
    op_emb = jnp.dot(oh.astype(bf), op_embed_ref[...].astype(bf),
                     preferred_element_type=jnp.float32)
    h = jnp.dot(op_emb.astype(bf), w_op_ref[...].astype(bf),
                preferred_element_type=jnp.float32)
    h = h + jnp.dot(a_bits.astype(bf), w_a_ref[...].astype(bf),
                    preferred_element_type=jnp.float32)
    h = h + jnp.dot(b_bits.astype(bf), w_b_ref[...].astype(bf),
                    preferred_element_type=jnp.float32)
    h = h + c_col * w_c_ref[...].astype(bf).astype(jnp.float32)
    h = h + b_in_ref[...]
    h_ref[...] = h

    rl = jnp.dot(h.astype(bf), wr_ref[...].astype(bf),
                 preferred_element_type=jnp.float32) + br_ref[...]
    rm = jnp.max(rl, axis=1, keepdims=True)
    re = jnp.exp(rl - rm)
    p = re / jnp.sum(re, axis=1, keepdims=True)
    ent = -jnp.sum(p * jnp.log(p + 1e-9), axis=1, keepdims=True)

    NEG = -1e30
    iota16 = lax.broadcasted_iota(jnp.int32, (BLK, NUM_TILES), 1)
    cur = rl
    tis, tvs = [], []
    for _ in range(K_TOP):
        mv = jnp.max(cur, axis=1, keepdims=True)
        idx = jnp.min(jnp.where(cur == mv, iota16, NUM_TILES),
                      axis=1, keepdims=True)
        tis.append(idx)
        tvs.append(mv)
        cur = jnp.where(iota16 == idx, NEG, cur)
    tile_idx = jnp.concatenate(tis, axis=1)
    top_vals = jnp.concatenate(tvs, axis=1)
    ge = jnp.exp(top_vals - top_vals[:, :1])
    gates = ge / jnp.sum(ge, axis=1, keepdims=True)
    tile_ref[...] = tile_idx
    gates_ref[...] = gates

    kl = jnp.dot(h.astype(bf), wk_ref[...].astype(bf),
                 preferred_element_type=jnp.float32) + bk_ref[...]
    km = jnp.max(kl, axis=1, keepdims=True)
    iota2k = lax.broadcasted_iota(jnp.int32, (BLK, TABLE_SIZE), 1)
    key_idx = jnp.min(jnp.where(kl == km, iota2k, TABLE_SIZE),
                      axis=1, keepdims=True)
    flat_ref[...] = tile_idx * TABLE_SIZE + key_idx

    v = v_ref[0]
    av = jnp.abs(v)
    s = av * (av - 1.0)
    tern_part = jnp.sum(jnp.sum(s * s, axis=1, keepdims=True),
                        axis=0, keepdims=True)

    cnt = jnp.sum((cur == NEG).astype(jnp.float32), axis=0, keepdims=True)

    @pl.when(i == 0)
    def _():
        psum_ref[...] = jnp.zeros_like(psum_ref)
        csum_ref[...] = jnp.zeros_like(csum_ref)
        esum_ref[...] = jnp.zeros_like(esum_ref)
        tsum_ref[...] = jnp.zeros_like(tsum_ref)

    psum_ref[...] += jnp.sum(p, axis=0, keepdims=True)
    csum_ref[...] += cnt
    esum_ref[...] += jnp.sum(ent, axis=0, keepdims=True)
    tsum_ref[...] += tern_part


def _head_kernel(h_ref, ws_ref, w1_ref, b1_ref, w2_ref, b2_ref, out_ref):
    ffn = h_ref[...] + ws_ref[...]
    bf = jnp.bfloat16
    hid = jnp.maximum(
        jnp.dot(ffn.astype(bf), w1_ref[...].astype(bf),
                preferred_element_type=jnp.float32)
        + b1_ref[...], 0.0)
    res = jnp.dot(hid.astype(bf), w2_ref[...].astype(bf),
                  preferred_element_type=jnp.float32) + b2_ref[...]
    out_ref[...] = 1.0 / (1.0 + jnp.exp(-res))


def _sc_gather_ws(fi_hbm, gb_hbm, tab_hbm, ws_hbm,
                  idx_all, g_all, rows0, rows1, acc0, acc1,
                  sem_a, sem_b, sem_o):
    wid = lax.axis_index("s") * NC + lax.axis_index("c")
    tok_base = wid * TOK_W

    pltpu.sync_copy(fi_hbm.at[pl.ds(tok_base * K_TOP, TOK_W * K_TOP)], idx_all)
    pltpu.sync_copy(gb_hbm.at[pl.ds(tok_base * K_TOP * L, TOK_W * K_TOP * L)],
                    g_all)

    def fire(ci, rows_ref, sem):
        pltpu.async_copy(tab_hbm.at[idx_all.at[pl.ds(ci * ROWS, ROWS)]],
                         rows_ref, sem)

    def drain_gather(rows_ref, sem):
        pltpu.make_async_copy(tab_hbm.at[pl.ds(0, ROWS)], rows_ref, sem).wait()

    def drain_store(acc_ref):
        pltpu.make_async_copy(ws_hbm.at[pl.ds(0, CH)], acc_ref, sem_o).wait()

    def compute(ci, rows_ref, acc_ref):
        gbase = ci * ROWS * L

        def tok_body(t, _t):
            r = t * K_TOP
            ga = g_all[pl.ds(gbase + (r + 0) * L, L)]
            gb = g_all[pl.ds(gbase + (r + 1) * L, L)]
            gc = g_all[pl.ds(gbase + (r + 2) * L, L)]
            gd = g_all[pl.ds(gbase + (r + 3) * L, L)]

            @plsc.parallel_loop(0, D_MODEL // L, 1, unroll=4)
            def d_body(j):
                sl = pl.ds(j * L, L)
                acc_ref[t, sl] = (ga * rows_ref[r, sl]
                                  + gb * rows_ref[r + 1, sl]
                                  + gc * rows_ref[r + 2, sl]
                                  + gd * rows_ref[r + 3, sl])

            return 0

        lax.fori_loop(0, CH, tok_body, 0)

    fire(0, rows0, sem_a)

    def pair_body(cp, _p):
        c0 = cp * 2
        fire(c0 + 1, rows1, sem_b)
        drain_gather(rows0, sem_a)

        @pl.when(cp > 0)
        def _():
            drain_store(acc0)

        compute(c0, rows0, acc0)
        pltpu.async_copy(acc0, ws_hbm.at[pl.ds(tok_base + c0 * CH, CH)], sem_o)

        @pl.when(cp + 1 < NCH // 2)
        def _():
            fire(c0 + 2, rows0, sem_a)

        drain_gather(rows1, sem_b)

        @pl.when(cp > 0)
        def _():
            drain_store(acc1)

        compute(c0 + 1, rows1, acc1)
        pltpu.async_copy(acc1, ws_hbm.at[pl.ds(tok_base + (c0 + 1) * CH, CH)],
                         sem_o)
        return 0

    lax.fori_loop(0, NCH // 2, pair_body, 0)
    drain_store(acc0)
    drain_store(acc1)


def _sc_ws(fi, gb, tab):
    return pl.kernel(
        _sc_gather_ws,
        out_type=jax.ShapeDtypeStruct((B, D_MODEL), jnp.float32),
        mesh=plsc.VectorSubcoreMesh(core_axis_name="c", subcore_axis_name="s"),
        scratch_types=[
            pltpu.VMEM((TOK_W * K_TOP,), jnp.int32),
            pltpu.VMEM((TOK_W * K_TOP * L,), jnp.float32),
            pltpu.VMEM((ROWS, D_MODEL), jnp.float32),
            pltpu.VMEM((ROWS, D_MODEL), jnp.float32),
            pltpu.VMEM((CH, D_MODEL), jnp.float32),
            pltpu.VMEM((CH, D_MODEL), jnp.float32),
            pltpu.SemaphoreType.DMA,
            pltpu.SemaphoreType.DMA,
            pltpu.SemaphoreType.DMA,
        ],
    )(fi, gb, tab)


def kernel(op_idx, a, b, c, op_embed, W_in, b_in, Wr, br, Wk, bk, V,
           W1, b1, W2, b2):
    op2 = op_idx.astype(jnp.int32)[:, None]
    a2 = a.astype(jnp.int32)[:, None]
    b2c = b.astype(jnp.int32)[:, None]
    c2 = c.astype(jnp.int32)[:, None]

    w_op = W_in[0:32]
    w_a = W_in[32:40]
    w_b = W_in[40:48]
    w_c = W_in[48:49]

    const2 = lambda shape: pl.BlockSpec(shape, lambda i: (0, 0))
    h, tile_idx, flat, gates, psum, csum, esum, tsum = pl.pallas_call(
        _front_kernel,
        grid=(GRID,),
        in_specs=[
            pl.BlockSpec((BLK, 1), lambda i: (i, 0)),
            pl.BlockSpec((BLK, 1), lambda i: (i, 0)),
            pl.BlockSpec((BLK, 1), lambda i: (i, 0)),
            pl.BlockSpec((BLK, 1), lambda i: (i, 0)),
            const2((N_OPS, 32)),
            const2((32, D_MODEL)),
            const2((8, D_MODEL)),
            const2((8, D_MODEL)),
            const2((1, D_MODEL)),
            const2((1, D_MODEL)),
            const2((D_MODEL, NUM_TILES)),
            const2((1, NUM_TILES)),
            const2((D_MODEL, TABLE_SIZE)),
            const2((1, TABLE_SIZE)),
            pl.BlockSpec((1, TABLE_SIZE, D_MODEL), lambda i: (i, 0, 0)),
        ],
        out_specs=[
            pl.BlockSpec((BLK, D_MODEL), lambda i: (i, 0)),
            pl.BlockSpec((BLK, K_TOP), lambda i: (i, 0)),
            pl.BlockSpec((BLK, K_TOP), lambda i: (i, 0)),
            pl.BlockSpec((BLK, K_TOP), lambda i: (i, 0)),
            const2((1, NUM_TILES)),
            const2((1, NUM_TILES)),
            const2((1, 1)),
            const2((1, 1)),
        ],
        out_shape=[
            jax.ShapeDtypeStruct((B, D_MODEL), jnp.float32),
            jax.ShapeDtypeStruct((B, K_TOP), jnp.int32),
            jax.ShapeDtypeStruct((B, K_TOP), jnp.int32),
            jax.ShapeDtypeStruct((B, K_TOP), jnp.float32),
            jax.ShapeDtypeStruct((1, NUM_TILES), jnp.float32),
            jax.ShapeDtypeStruct((1, NUM_TILES), jnp.float32),
            jax.ShapeDtypeStruct((1, 1), jnp.float32),
            jax.ShapeDtypeStruct((1, 1), jnp.float32),
        ],
        compiler_params=pltpu.CompilerParams(
            dimension_semantics=("arbitrary",)),
    )(op2, a2, b2c, c2, op_embed, w_op, w_a, w_b, w_c,
      b_in.reshape(1, D_MODEL), Wr, br.reshape(1, NUM_TILES),
      Wk, bk.reshape(1, TABLE_SIZE), V)

    fi = flat.reshape(B * K_TOP)
    gb = jnp.broadcast_to(gates[:, :, None], (B, K_TOP, L)).reshape(-1)
    tab = V.reshape(NUM_TILES * TABLE_SIZE, D_MODEL)
    ws = _sc_ws(fi, gb, tab)

    result = pl.pallas_call(
        _head_kernel,
        grid=(GRID,),
        in_specs=[
            pl.BlockSpec((BLK, D_MODEL), lambda i: (i, 0)),
            pl.BlockSpec((BLK, D_MODEL), lambda i: (i, 0)),
            const2((D_MODEL, 64)),
            const2((1, 64)),
            const2((64, 8)),
            const2((1, 8)),
        ],
        out_specs=pl.BlockSpec((BLK, 8), lambda i: (i, 0)),
        out_shape=jax.ShapeDtypeStruct((B, 8), jnp.float32),
    )(h, ws, W1, b1.reshape(1, 64), W2, b2.reshape(1, 8))

    total = float(NUM_TILES * TABLE_SIZE * D_MODEL)
    tern = TERNARY_W * (tsum[0, 0] / total)
    sparsity = SPARSITY_W * (esum[0, 0] / B)
    frac = csum[0] / B
    imp = psum[0] / B
    diversity = DIVERSITY_W * NUM_TILES * jnp.sum(frac * imp)
    aux = tern + sparsity + diversity
    return result, tile_idx, aux
